# baseline (speedup 1.0000x reference)
import functools

import jax
import jax.numpy as jnp
from jax import lax
from jax.experimental import pallas as pl
from jax.experimental.pallas import tpu as pltpu
from jax.experimental.pallas import tpu_sc as plsc

N_TOKENS = 32768
D = 768
H = 256
NE = 3

NC = 2
NS = 16
NW = NC * NS
LANES = 16
CHUNK = N_TOKENS // NW
PART_W = 6 * LANES

BLK_A = 8192
BLK_C = 8192


def _logits_body(q_ref, wg_ref, bg_ref, lg_ref):
    lgt = lax.dot_general(wg_ref[...], q_ref[...], (((0,), (1,)), ((), ())),
                          preferred_element_type=jnp.float32)
    lgt = lgt + bg_ref[...]
    lgt8 = jnp.concatenate([lgt, jnp.zeros((8 - NE, BLK_A), jnp.float32)], 0)
    lg_ref[...] = jnp.transpose(lgt8.reshape(8, BLK_A // 128, 128), (1, 0, 2))


def _logits_call(query, Wg, bg2):
    nblk = N_TOKENS // BLK_A
    return pl.pallas_call(
        _logits_body,
        grid=(nblk,),
        in_specs=[
            pl.BlockSpec((BLK_A, D), lambda i: (i, 0)),
            pl.BlockSpec((D, NE), lambda i: (0, 0)),
            pl.BlockSpec((NE, 1), lambda i: (0, 0)),
        ],
        out_specs=pl.BlockSpec((BLK_A // 128, 8, 128), lambda i: (i, 0, 0)),
        out_shape=jax.ShapeDtypeStruct((N_TOKENS // 128, 8, 128), jnp.float32),
    )(query, Wg, bg2)


SLAB = CHUNK * 8


def _gate_body(lg_hbm, sw_hbm, swd_hbm, part_hbm, lv, wv, wd, pv):
    wid = lax.axis_index("s") * NC + lax.axis_index("c")
    base = wid * CHUNK
    pltpu.sync_copy(lg_hbm.at[pl.ds(wid * SLAB, SLAB)], lv)

    fzero = jnp.zeros((LANES,), jnp.float32)
    fone = jnp.full((LANES,), 1.0, jnp.float32)

    def step(j, acc):
        i0, i1, i2, c0, c1, c2 = acc
        off = j * LANES
        ta = 1024 * (j // 8) + LANES * (j % 8)
        l0 = lv[pl.ds(ta, LANES)]
        l1 = lv[pl.ds(ta + 128, LANES)]
        l2 = lv[pl.ds(ta + 256, LANES)]

        j2 = (l2 <= l0) & (l2 <= l1)
        j1 = (~j2) & (l1 <= l0)
        j0 = ~(j2 | j1)

        m = jnp.maximum(l0, jnp.maximum(l1, l2))
        e0 = jnp.exp(l0 - m)
        e1 = jnp.exp(l1 - m)
        e2 = jnp.exp(l2 - m)
        rt = fone / (e0 + e1 + e2)

        z0 = jnp.where(j0, fzero, e0)
        z1 = jnp.where(j1, fzero, e1)
        z2 = jnp.where(j2, fzero, e2)
        rs = fone / (z0 + z1 + z2)
        w0 = z0 * rs
        w1 = z1 * rs
        w2 = z2 * rs
        wv[pl.ds(ta, LANES)] = w0
        wv[pl.ds(ta + 128, LANES)] = w1
        wv[pl.ds(ta + 256, LANES)] = w2
        wd[pl.ds(off, LANES)] = w0
        wd[pl.ds(CHUNK + off, LANES)] = w1
        wd[pl.ds(2 * CHUNK + off, LANES)] = w2

        return (i0 + e0 * rt, i1 + e1 * rt, i2 + e2 * rt,
                c0 + jnp.where(j0, fzero, fone),
                c1 + jnp.where(j1, fzero, fone),
                c2 + jnp.where(j2, fzero, fone))

    init = (fzero, fzero, fzero, fzero, fzero, fzero)
    acc = lax.fori_loop(0, CHUNK // LANES, step, init)
    for k in range(6):
        pv[pl.ds(k * LANES, LANES)] = acc[k]

    pltpu.sync_copy(wv, sw_hbm.at[pl.ds(wid * SLAB, SLAB)])
    for e in range(NE):
        pltpu.sync_copy(wd.at[pl.ds(e * CHUNK, CHUNK)],
                        swd_hbm.at[pl.ds(e * N_TOKENS + base, CHUNK)])
    pltpu.sync_copy(pv, part_hbm.at[wid])


def _gate_call(lgq_flat):
    mesh = plsc.VectorSubcoreMesh(
        core_axis_name="c", subcore_axis_name="s",
        num_cores=NC, num_subcores=NS)
    f = pl.kernel(
        _gate_body,
        out_type=[
            jax.ShapeDtypeStruct((N_TOKENS * 8,), jnp.float32),
            jax.ShapeDtypeStruct((NE * N_TOKENS,), jnp.float32),
            jax.ShapeDtypeStruct((NW, PART_W), jnp.float32),
        ],
        mesh=mesh,
        scratch_types=[
            pltpu.VMEM((SLAB,), jnp.float32),
            pltpu.VMEM((SLAB,), jnp.float32),
            pltpu.VMEM((NE * CHUNK,), jnp.float32),
            pltpu.VMEM((PART_W,), jnp.float32),
        ],
        compiler_params=pltpu.CompilerParams(needs_layout_passes=False),
    )
    return f(lgq_flat)


def _head_body(nblk, swt_ref, part_ref, bn_ref, ws_ref, bs_ref, wc_ref, bc_ref,
               w1_ref, b1_ref, gamma_ref, beta_ref, w2_ref, b2_ref,
               out_ref, loss_ref, e1_ref):
    i = pl.program_id(0)

    @pl.when(i == 0)
    def _prep():
        neural = bn_ref[...][None, :]
        symbolic = (0.5 * ws_ref[...] + bs_ref[...])[None, :]
        categorical = (0.5 * wc_ref[...] + bc_ref[...])[None, :]
        experts = jnp.concatenate([neural, symbolic, categorical], axis=0)
        e1_ref[...] = jnp.dot(experts, w1_ref[...],
                              preferred_element_type=jnp.float32)

        p = part_ref[...]
        inv_b = 1.0 / N_TOKENS
        loss = 0.0
        for e in range(NE):
            imp = jnp.sum(p[:, e * LANES:(e + 1) * LANES])
            cnt = jnp.sum(p[:, (NE + e) * LANES:(NE + e + 1) * LANES])
            loss += (imp * inv_b) * (cnt * inv_b)
        loss_ref[0] = NE * loss

    r3 = swt_ref[...][:, 0:NE, :]
    swt = jnp.transpose(r3, (1, 0, 2)).reshape(NE, BLK_C)
    hpre = lax.dot_general(swt, e1_ref[...], (((0,), (0,)), ((), ())),
                           preferred_element_type=jnp.float32)
    hpre = hpre + b1_ref[...][None, :]

    g = 0.5 * hpre * (1.0 + lax.erf(hpre * 0.7071067811865476))

    mu = jnp.mean(g, axis=1, keepdims=True)
    var = jnp.mean((g - mu) * (g - mu), axis=1, keepdims=True)
    hn = (g - mu) * lax.rsqrt(var + 1e-5)
    hn = hn * gamma_ref[...][None, :] + beta_ref[...][None, :]

    out = jnp.dot(hn.astype(jnp.bfloat16), w2_ref[...],
                  preferred_element_type=jnp.float32)
    out_ref[...] = out + b2_ref[...][None, :]


def _head_call(swt, part, bn, ws, bs, wc, bc, W1, b1, gamma, beta, W2, b2):
    nblk = N_TOKENS // BLK_C
    full = lambda shape: pl.BlockSpec(shape, lambda i: tuple(0 for _ in shape))
    return pl.pallas_call(
        functools.partial(_head_body, nblk),
        grid=(nblk,),
        in_specs=[
            pl.BlockSpec((BLK_C // 128, 8, 128), lambda i: (i, 0, 0)),
            full((NW, PART_W)), full((D,)), full((D,)), full((D,)),
            full((D,)), full((D,)), full((D, H)), full((H,)), full((H,)),
            full((H,)), full((H, D)), full((D,)),
        ],
        out_specs=[
            pl.BlockSpec((BLK_C, D), lambda i: (i, 0)),
            pl.BlockSpec(memory_space=pltpu.SMEM),
        ],
        out_shape=[
            jax.ShapeDtypeStruct((N_TOKENS, D), jnp.float32),
            jax.ShapeDtypeStruct((1,), jnp.float32),
        ],
        scratch_shapes=[pltpu.VMEM((NE, H), jnp.float32)],
    )(swt, part, bn, ws, bs, wc, bc, W1, b1, gamma, beta,
      W2.astype(jnp.bfloat16), b2)


def kernel(query, Wg, bg, Wn, bn, ws, bs, wc, bc, W1, b1, gamma, beta, W2, b2):
    lgq = _logits_call(query, Wg, bg.reshape(NE, 1))
    swq_flat, swd_flat, part = _gate_call(lgq.reshape(-1))
    swq = swq_flat.reshape(N_TOKENS // 128, 8, 128)
    out, loss = _head_call(swq, part, bn, ws, bs, wc, bc,
                           W1, b1, gamma, beta, W2, b2)
    return out, swd_flat.reshape(NE, N_TOKENS).T, loss[0]


# speedup vs baseline: 1.0417x; 1.0417x over previous
import functools

import jax
import jax.numpy as jnp
from jax import lax
from jax.experimental import pallas as pl
from jax.experimental.pallas import tpu as pltpu
from jax.experimental.pallas import tpu_sc as plsc

N_TOKENS = 32768
D = 768
H = 256
NE = 3

NC = 2
NS = 16
NW = NC * NS
LANES = 16
CHUNK = N_TOKENS // NW
PART_W = 6 * LANES

BLK_A = 4096
BLK_C = 4096


def _logits_body(q_ref, wg_ref, bg_ref, lg_ref):
    lgt = lax.dot_general(wg_ref[...], q_ref[...], (((0,), (1,)), ((), ())),
                          preferred_element_type=jnp.float32)
    lgt = lgt + bg_ref[...]
    lgt8 = jnp.concatenate([lgt, jnp.zeros((8 - NE, BLK_A), jnp.float32)], 0)
    lg_ref[...] = jnp.transpose(lgt8.reshape(8, BLK_A // 128, 128), (1, 0, 2))


def _logits_call(query, Wg, bg2):
    nblk = N_TOKENS // BLK_A
    return pl.pallas_call(
        _logits_body,
        grid=(nblk,),
        in_specs=[
            pl.BlockSpec((BLK_A, D), lambda i: (i, 0)),
            pl.BlockSpec((D, NE), lambda i: (0, 0)),
            pl.BlockSpec((NE, 1), lambda i: (0, 0)),
        ],
        out_specs=pl.BlockSpec((BLK_A // 128, 8, 128), lambda i: (i, 0, 0)),
        out_shape=jax.ShapeDtypeStruct((N_TOKENS // 128, 8, 128), jnp.float32),
    )(query, Wg, bg2)


SLAB = CHUNK * 8


def _gate_body(lg_hbm, sw_hbm, swd_hbm, part_hbm, lv, wv, wd, pv):
    wid = lax.axis_index("s") * NC + lax.axis_index("c")
    base = wid * CHUNK
    pltpu.sync_copy(lg_hbm.at[pl.ds(wid * SLAB, SLAB)], lv)

    fzero = jnp.zeros((LANES,), jnp.float32)
    fone = jnp.full((LANES,), 1.0, jnp.float32)

    def step(j, acc):
        i0, i1, i2, c0, c1, c2 = acc
        off = j * LANES
        ta = 1024 * (j // 8) + LANES * (j % 8)
        l0 = lv[pl.ds(ta, LANES)]
        l1 = lv[pl.ds(ta + 128, LANES)]
        l2 = lv[pl.ds(ta + 256, LANES)]

        j2 = (l2 <= l0) & (l2 <= l1)
        j1 = (~j2) & (l1 <= l0)
        j0 = ~(j2 | j1)

        m = jnp.maximum(l0, jnp.maximum(l1, l2))
        e0 = jnp.exp(l0 - m)
        e1 = jnp.exp(l1 - m)
        e2 = jnp.exp(l2 - m)
        rt = fone / (e0 + e1 + e2)

        z0 = jnp.where(j0, fzero, e0)
        z1 = jnp.where(j1, fzero, e1)
        z2 = jnp.where(j2, fzero, e2)
        rs = fone / (z0 + z1 + z2)
        w0 = z0 * rs
        w1 = z1 * rs
        w2 = z2 * rs
        wv[pl.ds(ta, LANES)] = w0
        wv[pl.ds(ta + 128, LANES)] = w1
        wv[pl.ds(ta + 256, LANES)] = w2
        wd[pl.ds(off, LANES)] = w0
        wd[pl.ds(CHUNK + off, LANES)] = w1
        wd[pl.ds(2 * CHUNK + off, LANES)] = w2

        return (i0 + e0 * rt, i1 + e1 * rt, i2 + e2 * rt,
                c0 + jnp.where(j0, fzero, fone),
                c1 + jnp.where(j1, fzero, fone),
                c2 + jnp.where(j2, fzero, fone))

    init = (fzero, fzero, fzero, fzero, fzero, fzero)
    acc = lax.fori_loop(0, CHUNK // LANES, step, init)
    for k in range(6):
        pv[pl.ds(k * LANES, LANES)] = acc[k]

    pltpu.sync_copy(wv, sw_hbm.at[pl.ds(wid * SLAB, SLAB)])
    for e in range(NE):
        pltpu.sync_copy(wd.at[pl.ds(e * CHUNK, CHUNK)],
                        swd_hbm.at[pl.ds(e * N_TOKENS + base, CHUNK)])
    pltpu.sync_copy(pv, part_hbm.at[wid])


def _gate_call(lgq_flat):
    mesh = plsc.VectorSubcoreMesh(
        core_axis_name="c", subcore_axis_name="s",
        num_cores=NC, num_subcores=NS)
    f = pl.kernel(
        _gate_body,
        out_type=[
            jax.ShapeDtypeStruct((N_TOKENS * 8,), jnp.float32),
            jax.ShapeDtypeStruct((NE * N_TOKENS,), jnp.float32),
            jax.ShapeDtypeStruct((NW, PART_W), jnp.float32),
        ],
        mesh=mesh,
        scratch_types=[
            pltpu.VMEM((SLAB,), jnp.float32),
            pltpu.VMEM((SLAB,), jnp.float32),
            pltpu.VMEM((NE * CHUNK,), jnp.float32),
            pltpu.VMEM((PART_W,), jnp.float32),
        ],
        compiler_params=pltpu.CompilerParams(needs_layout_passes=False),
    )
    return f(lgq_flat)


def _head_body(nblk, swt_ref, part_ref, bn_ref, ws_ref, bs_ref, wc_ref, bc_ref,
               w1_ref, b1_ref, gamma_ref, beta_ref, w2_ref, b2_ref,
               out_ref, loss_ref, e1_ref):
    i = pl.program_id(0)

    @pl.when(i == 0)
    def _prep():
        neural = bn_ref[...][None, :]
        symbolic = (0.5 * ws_ref[...] + bs_ref[...])[None, :]
        categorical = (0.5 * wc_ref[...] + bc_ref[...])[None, :]
        experts = jnp.concatenate([neural, symbolic, categorical], axis=0)
        e1_ref[...] = jnp.dot(experts, w1_ref[...],
                              preferred_element_type=jnp.float32)

        p = part_ref[...]
        inv_b = 1.0 / N_TOKENS
        loss = 0.0
        for e in range(NE):
            imp = jnp.sum(p[:, e * LANES:(e + 1) * LANES])
            cnt = jnp.sum(p[:, (NE + e) * LANES:(NE + e + 1) * LANES])
            loss += (imp * inv_b) * (cnt * inv_b)
        loss_ref[0] = NE * loss

    r3 = swt_ref[...][:, 0:NE, :]
    swt = jnp.transpose(r3, (1, 0, 2)).reshape(NE, BLK_C)
    hpre = lax.dot_general(swt, e1_ref[...], (((0,), (0,)), ((), ())),
                           preferred_element_type=jnp.float32)
    hpre = hpre + b1_ref[...][None, :]

    g = 0.5 * hpre * (1.0 + lax.erf(hpre * 0.7071067811865476))

    mu = jnp.mean(g, axis=1, keepdims=True)
    var = jnp.mean((g - mu) * (g - mu), axis=1, keepdims=True)
    hn = (g - mu) * lax.rsqrt(var + 1e-5)
    hn = hn * gamma_ref[...][None, :] + beta_ref[...][None, :]

    out = jnp.dot(hn.astype(jnp.bfloat16), w2_ref[...],
                  preferred_element_type=jnp.float32)
    out_ref[...] = out + b2_ref[...][None, :]


def _head_call(swt, part, bn, ws, bs, wc, bc, W1, b1, gamma, beta, W2, b2):
    nblk = N_TOKENS // BLK_C
    full = lambda shape: pl.BlockSpec(shape, lambda i: tuple(0 for _ in shape))
    return pl.pallas_call(
        functools.partial(_head_body, nblk),
        grid=(nblk,),
        in_specs=[
            pl.BlockSpec((BLK_C // 128, 8, 128), lambda i: (i, 0, 0)),
            full((NW, PART_W)), full((D,)), full((D,)), full((D,)),
            full((D,)), full((D,)), full((D, H)), full((H,)), full((H,)),
            full((H,)), full((H, D)), full((D,)),
        ],
        out_specs=[
            pl.BlockSpec((BLK_C, D), lambda i: (i, 0)),
            pl.BlockSpec(memory_space=pltpu.SMEM),
        ],
        out_shape=[
            jax.ShapeDtypeStruct((N_TOKENS, D), jnp.float32),
            jax.ShapeDtypeStruct((1,), jnp.float32),
        ],
        scratch_shapes=[pltpu.VMEM((NE, H), jnp.float32)],
    )(swt, part, bn, ws, bs, wc, bc, W1, b1, gamma, beta,
      W2.astype(jnp.bfloat16), b2)


def kernel(query, Wg, bg, Wn, bn, ws, bs, wc, bc, W1, b1, gamma, beta, W2, b2):
    lgq = _logits_call(query, Wg, bg.reshape(NE, 1))
    swq_flat, swd_flat, part = _gate_call(lgq.reshape(-1))
    swq = swq_flat.reshape(N_TOKENS // 128, 8, 128)
    out, loss = _head_call(swq, part, bn, ws, bs, wc, bc,
                           W1, b1, gamma, beta, W2, b2)
    return out, swd_flat.reshape(NE, N_TOKENS).T, loss[0]
